# baseline (device time: 37139 ns/iter reference)
import contextlib
import os

import jax
import jax.numpy as jnp
from jax import lax
from jax.experimental import pallas as pl
from jax.experimental.pallas import tpu as pltpu

M_BLOCK = 1024
HALF = 512
K = int(os.environ.get("RSRMS_K", "16"))
C = HALF // K
EPS = 1e-6
SCOPES = os.environ.get("RSRMS_SCOPES", "0") == "1"


def _scope(name):
    return jax.named_scope(name) if SCOPES else contextlib.nullcontext()


def kernel(partial, gamma):
    _, m_total, d = partial.shape
    g2d = gamma.reshape(1, d)

    def body(p_ref, g_ref, out_ref, x_recv, loc_buf, o_buf,
             x_send_sems, x_recv_sems, y_send_sems, y_recv_sems,
             loc_sems, outw_sems):
        my_x = lax.axis_index("x")
        my_y = lax.axis_index("y")
        other_x = 1 - my_x
        other_y = 1 - my_y

        with _scope("entry_barrier"):
            barrier_sem = pltpu.get_barrier_semaphore()
            for dev in ((other_x, my_y), (my_x, other_y)):
                pl.semaphore_signal(
                    barrier_sem, inc=1,
                    device_id=dev, device_id_type=pl.DeviceIdType.MESH,
                )
            pl.semaphore_wait(barrier_sem, 2)

        x_src_base = other_x * M_BLOCK + my_y * HALF
        x_rdmas = []
        with _scope("x_issue"):
            for i in range(K):
                r = pltpu.make_async_remote_copy(
                    src_ref=p_ref.at[0, pl.ds(x_src_base + i * C, C), :],
                    dst_ref=x_recv.at[i],
                    send_sem=x_send_sems.at[i],
                    recv_sem=x_recv_sems.at[i],
                    device_id=(other_x, my_y),
                    device_id_type=pl.DeviceIdType.MESH,
                )
                r.start()
                x_rdmas.append(r)

        loc_base = my_x * M_BLOCK + my_y * HALF
        loc_dmas = []
        with _scope("loc_issue"):
            for i in range(K):
                c = pltpu.make_async_copy(
                    p_ref.at[0, pl.ds(loc_base + i * C, C), :],
                    loc_buf.at[i],
                    loc_sems.at[i],
                )
                c.start()
                loc_dmas.append(c)

        y_rdmas = []
        out_dmas = []
        for i in range(K):
            with _scope(f"wait_xrecv#hop={i}"):
                x_rdmas[i].wait_recv()
                loc_dmas[i].wait()
            with _scope(f"compute#hop={i}"):
                s = loc_buf[i] + x_recv[i]
                ms = jnp.mean(s * s, axis=-1, keepdims=True)
                o_buf[i] = s * lax.rsqrt(ms + EPS) * g_ref[...]
            out_slice = pl.ds(my_y * HALF + i * C, C)
            with _scope(f"y_issue#hop={i}"):
                r = pltpu.make_async_remote_copy(
                    src_ref=o_buf.at[i],
                    dst_ref=out_ref.at[out_slice, :],
                    send_sem=y_send_sems.at[i],
                    recv_sem=y_recv_sems.at[i],
                    device_id=(my_x, other_y),
                    device_id_type=pl.DeviceIdType.MESH,
                )
                r.start()
                y_rdmas.append(r)
                w = pltpu.make_async_copy(
                    o_buf.at[i], out_ref.at[out_slice, :], outw_sems.at[i]
                )
                w.start()
                out_dmas.append(w)

        with _scope("drain"):
            for i in range(K):
                y_rdmas[i].wait_recv()
                out_dmas[i].wait()
            for i in range(K):
                x_rdmas[i].wait_send()
                y_rdmas[i].wait_send()

    return pl.pallas_call(
        body,
        out_shape=jax.ShapeDtypeStruct((M_BLOCK, d), jnp.float32),
        in_specs=[
            pl.BlockSpec(memory_space=pltpu.MemorySpace.HBM),
            pl.BlockSpec(memory_space=pltpu.MemorySpace.VMEM),
        ],
        out_specs=pl.BlockSpec(memory_space=pltpu.MemorySpace.HBM),
        scratch_shapes=[
            pltpu.VMEM((K, C, d), jnp.float32),
            pltpu.VMEM((K, C, d), jnp.float32),
            pltpu.VMEM((K, C, d), jnp.float32),
            pltpu.SemaphoreType.DMA((K,)),
            pltpu.SemaphoreType.DMA((K,)),
            pltpu.SemaphoreType.DMA((K,)),
            pltpu.SemaphoreType.DMA((K,)),
            pltpu.SemaphoreType.DMA((K,)),
            pltpu.SemaphoreType.DMA((K,)),
        ],
        compiler_params=pltpu.CompilerParams(collective_id=0),
    )(partial, g2d)


# device time: 27039 ns/iter; 1.3735x vs baseline; 1.3735x over previous
import contextlib
import os

import jax
import jax.numpy as jnp
from jax import lax
from jax.experimental import pallas as pl
from jax.experimental.pallas import tpu as pltpu

M_BLOCK = 1024
HALF = 512
EPS = 1e-6

if os.environ.get("RSRMS_SIZES"):
    SIZES = [int(s) for s in os.environ["RSRMS_SIZES"].split(",")]
else:
    SIZES = [HALF // int(os.environ.get("RSRMS_K", "4"))] * int(
        os.environ.get("RSRMS_K", "4")
    )
assert sum(SIZES) == HALF
OFFS = [sum(SIZES[:i]) for i in range(len(SIZES))]
K = len(SIZES)
SCOPES = os.environ.get("RSRMS_SCOPES", "0") == "1"
WIRE_DTYPE = jnp.float32 if os.environ.get("RSRMS_F32_WIRE") == "1" else jnp.bfloat16


def _scope(name):
    return jax.named_scope(name) if SCOPES else contextlib.nullcontext()


def kernel(partial, gamma):
    _, m_total, d = partial.shape

    def body(p_ref, g_ref, out_ref,
             xs_f32, xs_wire, loc_buf, x_recv, o_buf, o_wire, y_recv, yo_buf,
             x_send_sems, x_recv_sems, y_send_sems, y_recv_sems,
             xs_sem, loc_sem, outw_sems, yow_sems):
        my_x = lax.axis_index("x")
        my_y = lax.axis_index("y")
        other_x = 1 - my_x
        other_y = 1 - my_y

        x_src_base = other_x * M_BLOCK + my_y * HALF
        loc_base = my_x * M_BLOCK + my_y * HALF
        xs_dma = pltpu.make_async_copy(
            p_ref.at[0, pl.ds(x_src_base, HALF), :], xs_f32, xs_sem
        )
        xs_dma.start()
        loc_dma = pltpu.make_async_copy(
            p_ref.at[0, pl.ds(loc_base, HALF), :], loc_buf, loc_sem
        )
        loc_dma.start()

        with _scope("entry_barrier"):
            barrier_sem = pltpu.get_barrier_semaphore()
            for dev in ((other_x, my_y), (my_x, other_y)):
                pl.semaphore_signal(
                    barrier_sem, inc=1,
                    device_id=dev, device_id_type=pl.DeviceIdType.MESH,
                )
            pl.semaphore_wait(barrier_sem, 2)

        with _scope("convert_send"):
            xs_dma.wait()
            xs_wire[...] = xs_f32[...].astype(WIRE_DTYPE)

        x_rdmas = []
        with _scope("x_issue"):
            for i in range(K):
                sl = pl.ds(OFFS[i], SIZES[i])
                r = pltpu.make_async_remote_copy(
                    src_ref=xs_wire.at[sl, :],
                    dst_ref=x_recv.at[sl, :],
                    send_sem=x_send_sems.at[i],
                    recv_sem=x_recv_sems.at[i],
                    device_id=(other_x, my_y),
                    device_id_type=pl.DeviceIdType.MESH,
                )
                r.start()
                x_rdmas.append(r)

        loc_dma.wait()
        g = g_ref[...].reshape(1, d)

        y_rdmas = []
        out_dmas = []
        yo_dmas = []

        def process_y(j):
            sl = pl.ds(OFFS[j], SIZES[j])
            with _scope(f"y_store#hop={j}"):
                y_rdmas[j].wait_recv()
                yo_buf[sl, :] = y_recv[sl, :].astype(jnp.float32)
                w = pltpu.make_async_copy(
                    yo_buf.at[sl, :],
                    out_ref.at[pl.ds(other_y * HALF + OFFS[j], SIZES[j]), :],
                    yow_sems.at[j],
                )
                w.start()
                yo_dmas.append(w)

        for i in range(K):
            sl = pl.ds(OFFS[i], SIZES[i])
            with _scope(f"wait_xrecv#hop={i}"):
                x_rdmas[i].wait_recv()
            with _scope(f"compute#hop={i}"):
                s = loc_buf[sl, :] + x_recv[sl, :].astype(jnp.float32)
                ms = jnp.mean(s * s, axis=-1, keepdims=True)
                o = s * lax.rsqrt(ms + EPS) * g
                o_buf[sl, :] = o
                o_wire[sl, :] = o.astype(WIRE_DTYPE)
            with _scope(f"y_issue#hop={i}"):
                r = pltpu.make_async_remote_copy(
                    src_ref=o_wire.at[sl, :],
                    dst_ref=y_recv.at[sl, :],
                    send_sem=y_send_sems.at[i],
                    recv_sem=y_recv_sems.at[i],
                    device_id=(my_x, other_y),
                    device_id_type=pl.DeviceIdType.MESH,
                )
                r.start()
                y_rdmas.append(r)
                w = pltpu.make_async_copy(
                    o_buf.at[sl, :],
                    out_ref.at[pl.ds(my_y * HALF + OFFS[i], SIZES[i]), :],
                    outw_sems.at[i],
                )
                w.start()
                out_dmas.append(w)
            if i >= 1:
                process_y(i - 1)
        process_y(K - 1)

        with _scope("drain"):
            for i in range(K):
                out_dmas[i].wait()
                yo_dmas[i].wait()
            for i in range(K):
                x_rdmas[i].wait_send()
                y_rdmas[i].wait_send()

    return pl.pallas_call(
        body,
        out_shape=jax.ShapeDtypeStruct((M_BLOCK, d), jnp.float32),
        in_specs=[
            pl.BlockSpec(memory_space=pltpu.MemorySpace.HBM),
            pl.BlockSpec(memory_space=pltpu.MemorySpace.VMEM),
        ],
        out_specs=pl.BlockSpec(memory_space=pltpu.MemorySpace.HBM),
        scratch_shapes=[
            pltpu.VMEM((HALF, d), jnp.float32),
            pltpu.VMEM((HALF, d), WIRE_DTYPE),
            pltpu.VMEM((HALF, d), jnp.float32),
            pltpu.VMEM((HALF, d), WIRE_DTYPE),
            pltpu.VMEM((HALF, d), jnp.float32),
            pltpu.VMEM((HALF, d), WIRE_DTYPE),
            pltpu.VMEM((HALF, d), WIRE_DTYPE),
            pltpu.VMEM((HALF, d), jnp.float32),
            pltpu.SemaphoreType.DMA((K,)),
            pltpu.SemaphoreType.DMA((K,)),
            pltpu.SemaphoreType.DMA((K,)),
            pltpu.SemaphoreType.DMA((K,)),
            pltpu.SemaphoreType.DMA,
            pltpu.SemaphoreType.DMA,
            pltpu.SemaphoreType.DMA((K,)),
            pltpu.SemaphoreType.DMA((K,)),
        ],
        compiler_params=pltpu.CompilerParams(collective_id=0),
    )(partial, gamma)
